# baseline (speedup 1.0000x reference)
import functools

import jax
import jax.numpy as jnp
from jax import lax
from jax.experimental import pallas as pl
from jax.experimental.pallas import tpu as pltpu
from jax.experimental.pallas import tpu_sc as plsc

B, H, W, C = 32, 224, 224, 3
HW = H * W
D = 16
NC, NS, L = 2, 16, 16
RCHUNK = 8
PX = RCHUNK * W
NVREG = W // L
SEG = 112
GPC = PX // SEG
NCHUNK = H // RCHUNK

_mesh = plsc.VectorSubcoreMesh(core_axis_name="c", subcore_axis_name="s")


@functools.partial(
    pl.kernel,
    out_type=jax.ShapeDtypeStruct((B * HW * C,), jnp.float32),
    mesh=_mesh,
    compiler_params=pltpu.CompilerParams(
        needs_layout_passes=False, use_tc_tiling_on_sc=False),
    scratch_types=[
        pltpu.VMEM((NVREG, L), jnp.float32),
        pltpu.VMEM((H, L), jnp.float32),
        pltpu.VMEM((6, L), jnp.float32),
        pltpu.VMEM((GPC, SEG), jnp.int32),
        pltpu.VMEM((4, PX), jnp.float32),
        pltpu.VMEM((PX, D), jnp.float32),
        pltpu.VMEM((PX * C,), jnp.float32),
        pltpu.SemaphoreType.DMA,
    ],
)
def _sampler(quad_ref, thp_ref, ut_ref, rowu_ref, out_ref, ut_v, rowu_v,
             th_v, idxbuf, wbuf, taps, outbuf, sem):
    cid = lax.axis_index("c")
    sid = lax.axis_index("s")
    wid = sid * NC + cid
    pltpu.sync_copy(thp_ref.at[wid], th_v)
    pltpu.sync_copy(ut_ref, ut_v)
    pltpu.sync_copy(rowu_ref, rowu_v)

    av, bv, cv = th_v[0, :], th_v[1, :], th_v[2, :]
    dv, ev, fv = th_v[3, :], th_v[4, :], th_v[5, :]
    lane = lax.iota(jnp.int32, L)
    bbase = wid * HW
    zf = jnp.zeros((L,), jnp.float32)

    def flr(v):
        t = v.astype(jnp.int32)
        tf = t.astype(jnp.float32)
        return t - jnp.where(tf > v, 1, 0)

    def chunk_body(ch, carry):
        row0 = ch * RCHUNK

        def gen_row(r, carry2):
            uiv = rowu_v[row0 + r, :]
            rx = bv * uiv + cv
            ry = ev * uiv + fv
            for v in range(NVREG):
                uv = ut_v[v, :]
                xn = av * uv + rx
                yn = dv * uv + ry
                x = (0.5 * (xn + 1.0)) * jnp.float32(W - 1)
                y = (0.5 * (yn + 1.0)) * jnp.float32(H - 1)
                x0 = flr(x)
                y0 = flr(y)
                x0c = jnp.clip(x0, 0, W - 1)
                x1c = jnp.clip(x0 + 1, 0, W - 1)
                y0c = jnp.clip(y0, 0, H - 1)
                y1c = jnp.clip(y0 + 1, 0, H - 1)
                x0f = x0c.astype(jnp.float32)
                x1f = x1c.astype(jnp.float32)
                y0f = y0c.astype(jnp.float32)
                y1f = y1c.astype(jnp.float32)
                dx1 = x1f - x
                dx0 = x - x0f
                dy1 = y1f - y
                dy0 = y - y0f
                wa = dx1 * dy1
                wb = dx1 * dy0
                wc = dx0 * dy1
                wd = dx0 * dy0
                sx = x0c == x1c
                wa = wa + jnp.where(sx, wc, zf)
                wc = jnp.where(sx, zf, wc)
                wb = wb + jnp.where(sx, wd, zf)
                wd = jnp.where(sx, zf, wd)
                sy = y0c == y1c
                wa = wa + jnp.where(sy, wb, zf)
                wb = jnp.where(sy, zf, wb)
                wc = wc + jnp.where(sy, wd, zf)
                wd = jnp.where(sy, zf, wd)
                h = 0 if v < NVREG // 2 else 1
                col = L * v - SEG * h
                idxbuf[2 * r + h, pl.ds(col, L)] = bbase + y0c * W + x0c
                p0 = r * W + L * v
                wbuf[0, pl.ds(p0, L)] = wa
                wbuf[1, pl.ds(p0, L)] = wb
                wbuf[2, pl.ds(p0, L)] = wc
                wbuf[3, pl.ds(p0, L)] = wd
            return carry2

        lax.fori_loop(0, RCHUNK, gen_row, 0)

        descs = []
        for g in range(GPC):
            descs.append(pltpu.async_copy(
                quad_ref.at[idxbuf.at[g]],
                taps.at[pl.ds(g * SEG, SEG)],
                sem,
            ))
        for dsc in descs:
            dsc.wait()

        def use_row(r, carry2):
            for v in range(NVREG):
                p0 = r * W + L * v
                prow = lane + p0
                pout = prow * C
                wa = wbuf[0, pl.ds(p0, L)]
                wb = wbuf[1, pl.ds(p0, L)]
                wc = wbuf[2, pl.ds(p0, L)]
                wd = wbuf[3, pl.ds(p0, L)]
                for c in range(C):
                    cc = jnp.full((L,), c, jnp.int32)
                    va = plsc.load_gather(taps, [prow, cc])
                    vb = plsc.load_gather(taps, [prow, cc + C])
                    vc = plsc.load_gather(taps, [prow, cc + 2 * C])
                    vd = plsc.load_gather(taps, [prow, cc + 3 * C])
                    o = wa * va + wb * vb + wc * vc + wd * vd
                    plsc.store_scatter(outbuf, [pout + c], o)
            return carry2

        lax.fori_loop(0, RCHUNK, use_row, 0)
        pltpu.sync_copy(
            outbuf, out_ref.at[pl.ds(wid * HW * C + ch * PX * C, PX * C)])
        return carry

    lax.fori_loop(0, NCHUNK, chunk_body, 0)


def _rne_bf16(v):
    u = jax.lax.bitcast_convert_type(v, jnp.uint32)
    r = (u + jnp.uint32(0x7FFF) + ((u >> 16) & jnp.uint32(1)))
    r = r & jnp.uint32(0xFFFF0000)
    return jax.lax.bitcast_convert_type(r, jnp.float32)


def kernel(inputs):
    theta = inputs[:, :6]
    img = jnp.reshape(inputs[:, 6:], (B, H, W, C))
    sx = jnp.concatenate([img[:, :, 1:, :], img[:, :, -1:, :]], axis=2)
    sy = jnp.concatenate([img[:, 1:, :, :], img[:, -1:, :, :]], axis=1)
    sxy = jnp.concatenate([sy[:, :, 1:, :], sy[:, :, -1:, :]], axis=2)
    pad = jnp.zeros((B, H, W, D - 4 * C), jnp.float32)
    quad = jnp.concatenate([img, sy, sx, sxy, pad], axis=-1)
    quad = jnp.reshape(quad, (B * HW, D))

    thp = _rne_bf16(theta)
    thp = jnp.broadcast_to(thp[:, :, None], (B, 6, L))
    u = _rne_bf16(jnp.linspace(-1.0, 1.0, W))
    ut = jnp.reshape(u, (NVREG, L))
    rowu = jnp.broadcast_to(u[:, None], (H, L))
    out = _sampler(quad, thp, ut, rowu)
    return jnp.reshape(out, (B, H, W, C))
